# baseline (speedup 1.0000x reference)
import functools

import jax
import jax.numpy as jnp
from jax import lax
from jax.experimental import pallas as pl
from jax.experimental.pallas import tpu as pltpu
from jax.experimental.pallas import tpu_sc as plsc

_N = 10000
_E = 320000
_F = 128
_H = 8
_D = 16
_FC = _F + 16
_NC = 2
_NS = 16
_NW = _NC * _NS
_EPW = _E // _NW
_B = 40
_NB = _EPW // _B
_NP = 10240
_RPT = _NP // _NS


def _proj_body(x_ref, wg_ref, ps_ref, pd_ref, hcat_ref, ad_ref):
    h = jnp.dot(x_ref[...], wg_ref[...], preferred_element_type=jnp.float32)
    hcat_ref[:, 0:_F] = h
    hcat_ref[:, _F:_FC] = jnp.dot(h, ps_ref[...],
                                  preferred_element_type=jnp.float32)
    ad_ref[...] = jnp.dot(h, pd_ref[...], preferred_element_type=jnp.float32)


_proj = pl.pallas_call(
    _proj_body,
    out_shape=(
        jax.ShapeDtypeStruct((_N, _FC), jnp.float32),
        jax.ShapeDtypeStruct((_N, _D), jnp.float32),
    ),
)


def _sc_body(src_hbm, dst_hbm, hcat_hbm, adst_hbm, pout_hbm,
             sidx0, didx0, sidx1, didx1, hrow0, brow0, hrow1, brow1,
             msgw0, msgw1, sdidx0, sdidx1,
             acc_sh, semi0, semi1, semg0, semg1, sems0, sems1):
    cid = lax.axis_index("c")
    sid = lax.axis_index("s")
    wid = sid * _NC + cid
    bufs = ((sidx0, didx0, hrow0, brow0, msgw0, semi0, semg0, sdidx0, sems0),
            (sidx1, didx1, hrow1, brow1, msgw1, semi1, semg1, sdidx1, sems1))

    zeros16 = jnp.zeros((16,), jnp.float32)

    def zrow(r, carry):
        for j in range(_FC // 16):
            msgw0[r, pl.ds(j * 16, 16)] = zeros16
        return carry

    lax.fori_loop(0, _B, zrow, 0)
    r0 = sid * _RPT
    for k in range(_RPT // _B):
        pltpu.sync_copy(msgw0, acc_sh.at[pl.ds(r0 + k * _B, _B)])
    plsc.subcore_barrier()

    lane = lax.iota(jnp.int32, 16)
    headmask = lane < _H
    base0 = wid * _EPW

    def fire_idx(it, p):
        base = base0 + it * _B
        pltpu.async_copy(src_hbm.at[pl.ds(base, _B)], bufs[p][0], bufs[p][5])
        pltpu.async_copy(dst_hbm.at[pl.ds(base, _B)], bufs[p][1], bufs[p][5])

    def wait_idx(it, p):
        base = base0 + it * _B
        pltpu.make_async_copy(src_hbm.at[pl.ds(base, _B)], bufs[p][0],
                              bufs[p][5]).wait()
        pltpu.make_async_copy(dst_hbm.at[pl.ds(base, _B)], bufs[p][1],
                              bufs[p][5]).wait()

    def fire_gather(p):
        pltpu.async_copy(hcat_hbm.at[bufs[p][0]], bufs[p][2], bufs[p][6])
        pltpu.async_copy(adst_hbm.at[bufs[p][1]], bufs[p][3], bufs[p][6])

    def wait_gather(p):
        pltpu.make_async_copy(hcat_hbm.at[bufs[p][0]], bufs[p][2],
                              bufs[p][6]).wait()
        pltpu.make_async_copy(adst_hbm.at[bufs[p][1]], bufs[p][3],
                              bufs[p][6]).wait()

    splat_idx = [jnp.full((16,), hd, jnp.int32) for hd in range(_H)]

    def compute(p):
        hrow, brow, msgw = bufs[p][2], bufs[p][3], bufs[p][4]

        @plsc.parallel_loop(0, _B, 1, unroll=4)
        def edge(i):
            a = hrow[i, pl.ds(_F, 16)]
            b = brow[i, :]
            e = a + b
            e = jnp.where(e >= 0.0, e, e * 0.2)
            w = jnp.where(headmask, jnp.exp(e), 0.0)
            msgw[i, pl.ds(_F, 16)] = w
            for hd in range(_H):
                ws = jnp.take_along_axis(w, splat_idx[hd], axis=0)
                msgw[i, pl.ds(hd * 16, 16)] = hrow[i, pl.ds(hd * 16, 16)] * ws

    def fire_scatter(p):
        didx, sdidx = bufs[p][1], bufs[p][7]
        for off in (0, 16, _B - 16):
            sdidx[pl.ds(off, 16)] = didx[pl.ds(off, 16)]
        pltpu.async_copy(bufs[p][4], acc_sh.at[sdidx], bufs[p][8], add=True)

    def wait_scatter(p):
        pltpu.make_async_copy(bufs[p][4], acc_sh.at[bufs[p][7]],
                              bufs[p][8]).wait()

    fire_idx(0, 0)
    wait_idx(0, 0)
    fire_gather(0)
    fire_idx(1, 1)
    last = _NB // 2 - 1

    def paired(g, carry):
        ite = 2 * g
        wait_idx(ite + 1, 1)
        fire_gather(1)
        wait_gather(0)

        compute(0)

        @pl.when(g > 0)
        def _():
            wait_scatter(1)

        fire_scatter(0)

        @pl.when(g < last)
        def _():
            fire_idx(ite + 2, 0)

        @pl.when(g < last)
        def _():
            wait_idx(ite + 2, 0)
            fire_gather(0)

        wait_gather(1)
        compute(1)
        wait_scatter(0)
        fire_scatter(1)

        @pl.when(g < last)
        def _():
            fire_idx(ite + 3, 1)

        return carry

    lax.fori_loop(0, _NB // 2, paired, 0)
    wait_scatter(1)

    plsc.subcore_barrier()
    pltpu.sync_copy(acc_sh.at[pl.ds(r0, _RPT)],
                    pout_hbm.at[cid, pl.ds(r0, _RPT)])


_sc_agg = functools.partial(
    pl.kernel,
    out_type=jax.ShapeDtypeStruct((_NC, _NP, _FC), jnp.float32),
    mesh=plsc.VectorSubcoreMesh(core_axis_name="c", subcore_axis_name="s"),
    compiler_params=pltpu.CompilerParams(use_tc_tiling_on_sc=False),
    scratch_types=[
        pltpu.VMEM((_B,), jnp.int32),
        pltpu.VMEM((_B,), jnp.int32),
        pltpu.VMEM((_B,), jnp.int32),
        pltpu.VMEM((_B,), jnp.int32),
        pltpu.VMEM((_B, _FC), jnp.float32),
        pltpu.VMEM((_B, _D), jnp.float32),
        pltpu.VMEM((_B, _FC), jnp.float32),
        pltpu.VMEM((_B, _D), jnp.float32),
        pltpu.VMEM((_B, _FC), jnp.float32),
        pltpu.VMEM((_B, _FC), jnp.float32),
        pltpu.VMEM((_B,), jnp.int32),
        pltpu.VMEM((_B,), jnp.int32),
        pltpu.VMEM_SHARED((_NP, _FC), jnp.float32),
        pltpu.SemaphoreType.DMA,
        pltpu.SemaphoreType.DMA,
        pltpu.SemaphoreType.DMA,
        pltpu.SemaphoreType.DMA,
        pltpu.SemaphoreType.DMA,
        pltpu.SemaphoreType.DMA,
    ],
)(_sc_body)


def _merge_body(p_ref, s_ref, bias_ref, g_ref, b_ref, o_ref):
    pc = p_ref[0, 0:_N, :] + p_ref[1, 0:_N, :]
    out = pc[:, 0:_F]
    den16 = pc[:, _F:_FC]
    den = jnp.dot(den16, s_ref[...], preferred_element_type=jnp.float32)
    den = jnp.where(den == 0.0, 1.0, den)
    y = out / den + bias_ref[...]
    mean = jnp.mean(y, axis=0, keepdims=True)
    var = jnp.mean((y - mean) ** 2, axis=0, keepdims=True)
    yn = (y - mean) / jnp.sqrt(var + 1e-5) * g_ref[...] + b_ref[...]
    o_ref[...] = jnp.where(yn > 0.0, yn, jnp.exp(yn) - 1.0)


_merge = pl.pallas_call(
    _merge_body,
    out_shape=jax.ShapeDtypeStruct((_N, _F), jnp.float32),
)


def kernel(x, edge_index, W_lin, b_lin, W_gat, att_src, att_dst, bias_gat,
           bn_gamma, bn_beta):
    src = edge_index[0]
    dst = edge_index[1]
    oh = jnp.eye(_H, _D, dtype=jnp.float32)
    ps = (att_src[:, :, None] * oh[:, None, :]).reshape(_F, _D)
    pd = (att_dst[:, :, None] * oh[:, None, :]).reshape(_F, _D)
    sexp = (jnp.arange(_F)[None, :] // _D ==
            jnp.arange(_D)[:, None]).astype(jnp.float32)

    hcat, adst = _proj(x, W_gat, ps, pd)
    pout = _sc_agg(src, dst, hcat, adst)
    return _merge(pout, sexp, bias_gat.reshape(1, _F),
                  bn_gamma.reshape(1, _F), bn_beta.reshape(1, _F))


# speedup vs baseline: 1.0388x; 1.0388x over previous
import functools

import jax
import jax.numpy as jnp
from jax import lax
from jax.experimental import pallas as pl
from jax.experimental.pallas import tpu as pltpu
from jax.experimental.pallas import tpu_sc as plsc

_N = 10000
_E = 320000
_F = 128
_H = 8
_D = 16
_FC = _F + 16
_FX = _F + 32
_NC = 2
_NS = 16
_NW = _NC * _NS
_EPW = _E // _NW
_B = 80
_NB = _EPW // _B
_NP = 10240
_RPT = _NP // _NS


def _proj_body(x_ref, wg_ref, ps_ref, pd_ref, p1_ref, p2_ref, hx_ref, ad_ref):
    h = jnp.dot(x_ref[...], wg_ref[...], preferred_element_type=jnp.float32)
    hx_ref[:, 0:_F] = jnp.dot(
        h, p1_ref[...], preferred_element_type=jnp.float32
    ).astype(jnp.bfloat16)
    a16 = jnp.dot(h, ps_ref[...], preferred_element_type=jnp.float32)
    hx_ref[:, _F:_FX] = jnp.dot(
        a16, p2_ref[...], preferred_element_type=jnp.float32
    ).astype(jnp.bfloat16)
    ad_ref[...] = jnp.dot(h, pd_ref[...], preferred_element_type=jnp.float32)


_proj = pl.pallas_call(
    _proj_body,
    out_shape=(
        jax.ShapeDtypeStruct((_N, _FX), jnp.bfloat16),
        jax.ShapeDtypeStruct((_N, _D), jnp.float32),
    ),
)


def _sc_body(src_hbm, dst_hbm, hx_hbm, adst_hbm, pout_hbm,
             sidx0, didx0, sidx1, didx1, hrow0, brow0, hrow1, brow1,
             msgw0, acc_sh, semi0, semi1, semg0, semg1):
    cid = lax.axis_index("c")
    sid = lax.axis_index("s")
    wid = sid * _NC + cid
    bufs = ((sidx0, didx0, hrow0, brow0, msgw0, semi0, semg0),
            (sidx1, didx1, hrow1, brow1, msgw0, semi1, semg1))

    zeros16 = jnp.zeros((16,), jnp.float32)

    def zrow(r, carry):
        for j in range(_FC // 16):
            msgw0[r, pl.ds(j * 16, 16)] = zeros16
        return carry

    lax.fori_loop(0, _B, zrow, 0)
    r0 = sid * _RPT
    for k in range(_RPT // _B):
        pltpu.sync_copy(msgw0, acc_sh.at[pl.ds(r0 + k * _B, _B)])
    plsc.subcore_barrier()

    lane = lax.iota(jnp.int32, 16)
    headmask = lane < _H
    base0 = wid * _EPW

    def fire_idx(it, p):
        base = base0 + it * _B
        pltpu.async_copy(src_hbm.at[pl.ds(base, _B)], bufs[p][0], bufs[p][5])
        pltpu.async_copy(dst_hbm.at[pl.ds(base, _B)], bufs[p][1], bufs[p][5])

    def wait_idx(it, p):
        base = base0 + it * _B
        pltpu.make_async_copy(src_hbm.at[pl.ds(base, _B)], bufs[p][0],
                              bufs[p][5]).wait()
        pltpu.make_async_copy(dst_hbm.at[pl.ds(base, _B)], bufs[p][1],
                              bufs[p][5]).wait()

    def fire_gather(p):
        pltpu.async_copy(hx_hbm.at[bufs[p][0]], bufs[p][2], bufs[p][6])
        pltpu.async_copy(adst_hbm.at[bufs[p][1]], bufs[p][3], bufs[p][6])

    def wait_gather(p):
        pltpu.make_async_copy(hx_hbm.at[bufs[p][0]], bufs[p][2],
                              bufs[p][6]).wait()
        pltpu.make_async_copy(adst_hbm.at[bufs[p][1]], bufs[p][3],
                              bufs[p][6]).wait()

    splat_idx = [jnp.full((16,), hd, jnp.int32) for hd in range(_H)]

    def compute_scatter(p):
        hrow, brow, msgw = bufs[p][2], bufs[p][3], bufs[p][4]

        @plsc.parallel_loop(0, _B, 1, unroll=4)
        def edge(i):
            av = hrow[i, pl.ds(_F, 32)]
            a, _ = plsc.unpack(av, format=plsc.PackFormat.INTERLEAVED)
            b = brow[i, :]
            e = a + b
            e = jnp.where(e >= 0.0, e, e * 0.2)
            w = jnp.where(headmask, jnp.exp(e), 0.0)
            msgw[i, pl.ds(_F, 16)] = w
            for j in range(_H // 2):
                hv = hrow[i, pl.ds(j * 32, 32)]
                h0, h1 = plsc.unpack(hv, format=plsc.PackFormat.INTERLEAVED)
                w0 = jnp.take_along_axis(w, splat_idx[2 * j], axis=0)
                w1 = jnp.take_along_axis(w, splat_idx[2 * j + 1], axis=0)
                msgw[i, pl.ds(j * 32, 16)] = h0 * w0
                msgw[i, pl.ds(j * 32 + 16, 16)] = h1 * w1

        pltpu.sync_copy(msgw, acc_sh.at[bufs[p][1]], add=True)

    fire_idx(0, 0)
    wait_idx(0, 0)
    fire_gather(0)
    fire_idx(1, 1)

    def paired(g, carry):
        ite = 2 * g
        wait_idx(ite + 1, 1)
        fire_gather(1)
        wait_gather(0)
        compute_scatter(0)
        fire_idx(ite + 2, 0)
        wait_idx(ite + 2, 0)
        fire_gather(0)
        wait_gather(1)
        compute_scatter(1)

        @pl.when(g < (_NB - 1) // 2 - 1)
        def _():
            fire_idx(ite + 3, 1)

        return carry

    lax.fori_loop(0, (_NB - 1) // 2, paired, 0)
    wait_gather(0)
    compute_scatter(0)

    plsc.subcore_barrier()
    pltpu.sync_copy(acc_sh.at[pl.ds(r0, _RPT)],
                    pout_hbm.at[cid, pl.ds(r0, _RPT)])


_sc_agg = functools.partial(
    pl.kernel,
    out_type=jax.ShapeDtypeStruct((_NC, _NP, _FC), jnp.float32),
    mesh=plsc.VectorSubcoreMesh(core_axis_name="c", subcore_axis_name="s"),
    compiler_params=pltpu.CompilerParams(use_tc_tiling_on_sc=False,
                                         needs_layout_passes=False),
    scratch_types=[
        pltpu.VMEM((_B,), jnp.int32),
        pltpu.VMEM((_B,), jnp.int32),
        pltpu.VMEM((_B,), jnp.int32),
        pltpu.VMEM((_B,), jnp.int32),
        pltpu.VMEM((_B, _FX), jnp.bfloat16),
        pltpu.VMEM((_B, _D), jnp.float32),
        pltpu.VMEM((_B, _FX), jnp.bfloat16),
        pltpu.VMEM((_B, _D), jnp.float32),
        pltpu.VMEM((_B, _FC), jnp.float32),
        pltpu.VMEM_SHARED((_NP, _FC), jnp.float32),
        pltpu.SemaphoreType.DMA,
        pltpu.SemaphoreType.DMA,
        pltpu.SemaphoreType.DMA,
        pltpu.SemaphoreType.DMA,
    ],
)(_sc_body)


def _merge_body(p_ref, s_ref, bias_ref, g_ref, b_ref, o_ref):
    pc = p_ref[0, 0:_N, :] + p_ref[1, 0:_N, :]
    out = pc[:, 0:_F]
    den16 = pc[:, _F:_FC]
    den = jnp.dot(den16, s_ref[...], preferred_element_type=jnp.float32)
    den = jnp.where(den == 0.0, 1.0, den)
    y = out / den + bias_ref[...]
    mean = jnp.mean(y, axis=0, keepdims=True)
    var = jnp.mean((y - mean) ** 2, axis=0, keepdims=True)
    yn = (y - mean) / jnp.sqrt(var + 1e-5) * g_ref[...] + b_ref[...]
    o_ref[...] = jnp.where(yn > 0.0, yn, jnp.exp(yn) - 1.0)


_merge = pl.pallas_call(
    _merge_body,
    out_shape=jax.ShapeDtypeStruct((_N, _F), jnp.float32),
)


def kernel(x, edge_index, W_lin, b_lin, W_gat, att_src, att_dst, bias_gat,
           bn_gamma, bn_beta):
    src = edge_index[0]
    dst = edge_index[1]
    oh = jnp.eye(_H, _D, dtype=jnp.float32)
    ps = (att_src[:, :, None] * oh[:, None, :]).reshape(_F, _D)
    pd = (att_dst[:, :, None] * oh[:, None, :]).reshape(_F, _D)
    csrc = jnp.arange(_F)
    cdst = (csrc // 32) * 32 + (csrc % _D) * 2 + (csrc // _D) % 2
    p1 = (jnp.arange(_F)[None, :] == cdst[:, None]).astype(jnp.float32)
    p2 = (jnp.arange(32)[None, :] ==
          (2 * jnp.arange(_D))[:, None]).astype(jnp.float32)
    sexp = (jnp.arange(_F)[None, :] // _D ==
            jnp.arange(_D)[:, None]).astype(jnp.float32)

    hx, adst = _proj(x, W_gat, ps, pd, p1, p2)
    pout = _sc_agg(src, dst, hx, adst)
    return _merge(pout, sexp, bias_gat.reshape(1, _F),
                  bn_gamma.reshape(1, _F), bn_beta.reshape(1, _F))


# speedup vs baseline: 1.1377x; 1.0952x over previous
import functools

import jax
import jax.numpy as jnp
from jax import lax
from jax.experimental import pallas as pl
from jax.experimental.pallas import tpu as pltpu
from jax.experimental.pallas import tpu_sc as plsc

_N = 10000
_E = 320000
_F = 128
_H = 8
_D = 16
_FC = _F + 16
_NC = 2
_NS = 16
_NW = _NC * _NS
_EPW = _E // _NW
_B = 80
_NB = _EPW // _B
_NP = 10240
_RPT = _NP // _NS


def _proj_body(x_ref, wg_ref, ps_ref, pd_ref, hcat_ref, ad_ref):
    h = jnp.dot(x_ref[...], wg_ref[...], preferred_element_type=jnp.float32)
    hcat_ref[:, 0:_F] = h
    hcat_ref[:, _F:_FC] = jnp.dot(h, ps_ref[...],
                                  preferred_element_type=jnp.float32)
    ad_ref[...] = jnp.dot(h, pd_ref[...], preferred_element_type=jnp.float32)


_proj = pl.pallas_call(
    _proj_body,
    out_shape=(
        jax.ShapeDtypeStruct((_N, _FC), jnp.float32),
        jax.ShapeDtypeStruct((_N, _D), jnp.float32),
    ),
)


def _sc_body(ei_hbm, hcat_hbm, adst_hbm, pout_hbm,
             idx0, idx1, hrow0, brow0, hrow1, brow1,
             msgw0, acc_sh, semi0, semi1, semg0, semg1):
    cid = lax.axis_index("c")
    sid = lax.axis_index("s")
    wid = sid * _NC + cid
    bufs = ((idx0, hrow0, brow0, msgw0, semi0, semg0),
            (idx1, hrow1, brow1, msgw0, semi1, semg1))

    zeros16 = jnp.zeros((16,), jnp.float32)

    def zrow(r, carry):
        for j in range(_FC // 16):
            msgw0[r, pl.ds(j * 16, 16)] = zeros16
        return carry

    lax.fori_loop(0, _B, zrow, 0)
    r0 = sid * _RPT
    for k in range(_RPT // _B):
        pltpu.sync_copy(msgw0, acc_sh.at[pl.ds(r0 + k * _B, _B)])
    plsc.subcore_barrier()

    lane = lax.iota(jnp.int32, 16)
    headmask = lane < _H
    base0 = wid * _EPW

    def fire_idx(it, p):
        base = base0 + it * _B
        pltpu.async_copy(ei_hbm.at[:, pl.ds(base, _B)], bufs[p][0],
                         bufs[p][4])

    def wait_idx(it, p):
        base = base0 + it * _B
        pltpu.make_async_copy(ei_hbm.at[:, pl.ds(base, _B)], bufs[p][0],
                              bufs[p][4]).wait()

    def fire_gather(p):
        pltpu.async_copy(hcat_hbm.at[bufs[p][0].at[0]], bufs[p][1],
                         bufs[p][5])
        pltpu.async_copy(adst_hbm.at[bufs[p][0].at[1]], bufs[p][2],
                         bufs[p][5])

    def wait_gather(p):
        pltpu.make_async_copy(hcat_hbm.at[bufs[p][0].at[0]], bufs[p][1],
                              bufs[p][5]).wait()
        pltpu.make_async_copy(adst_hbm.at[bufs[p][0].at[1]], bufs[p][2],
                              bufs[p][5]).wait()

    splat_idx = [jnp.full((16,), hd, jnp.int32) for hd in range(_H)]

    def compute_scatter(p):
        hrow, brow, msgw = bufs[p][1], bufs[p][2], bufs[p][3]

        @plsc.parallel_loop(0, _B, 1, unroll=4)
        def edge(i):
            a = hrow[i, pl.ds(_F, 16)]
            b = brow[i, :]
            e = a + b
            e = jnp.where(e >= 0.0, e, e * 0.2)
            w = jnp.where(headmask, jnp.exp(e), 0.0)
            msgw[i, pl.ds(_F, 16)] = w
            for hd in range(_H):
                ws = jnp.take_along_axis(w, splat_idx[hd], axis=0)
                msgw[i, pl.ds(hd * 16, 16)] = hrow[i, pl.ds(hd * 16, 16)] * ws

        pltpu.sync_copy(msgw, acc_sh.at[bufs[p][0].at[1]], add=True)

    fire_idx(0, 0)
    wait_idx(0, 0)
    fire_gather(0)
    fire_idx(1, 1)

    def paired(g, carry):
        ite = 2 * g
        wait_idx(ite + 1, 1)
        fire_gather(1)
        wait_gather(0)
        compute_scatter(0)
        fire_idx(ite + 2, 0)
        wait_idx(ite + 2, 0)
        fire_gather(0)
        wait_gather(1)
        compute_scatter(1)

        @pl.when(g < (_NB - 1) // 2 - 1)
        def _():
            fire_idx(ite + 3, 1)

        return carry

    lax.fori_loop(0, (_NB - 1) // 2, paired, 0)
    wait_gather(0)
    compute_scatter(0)

    plsc.subcore_barrier()
    pltpu.sync_copy(acc_sh.at[pl.ds(r0, _RPT)],
                    pout_hbm.at[cid, pl.ds(r0, _RPT)])


_sc_agg = functools.partial(
    pl.kernel,
    out_type=jax.ShapeDtypeStruct((_NC, _NP, _FC), jnp.float32),
    mesh=plsc.VectorSubcoreMesh(core_axis_name="c", subcore_axis_name="s"),
    compiler_params=pltpu.CompilerParams(use_tc_tiling_on_sc=False),
    scratch_types=[
        pltpu.VMEM((2, _B), jnp.int32),
        pltpu.VMEM((2, _B), jnp.int32),
        pltpu.VMEM((_B, _FC), jnp.float32),
        pltpu.VMEM((_B, _D), jnp.float32),
        pltpu.VMEM((_B, _FC), jnp.float32),
        pltpu.VMEM((_B, _D), jnp.float32),
        pltpu.VMEM((_B, _FC), jnp.float32),
        pltpu.VMEM_SHARED((_NP, _FC), jnp.float32),
        pltpu.SemaphoreType.DMA,
        pltpu.SemaphoreType.DMA,
        pltpu.SemaphoreType.DMA,
        pltpu.SemaphoreType.DMA,
    ],
)(_sc_body)


def _merge_body(p_ref, s_ref, bias_ref, g_ref, b_ref, o_ref):
    pc = p_ref[0, 0:_N, :] + p_ref[1, 0:_N, :]
    out = pc[:, 0:_F]
    den16 = pc[:, _F:_FC]
    den = jnp.dot(den16, s_ref[...], preferred_element_type=jnp.float32)
    den = jnp.where(den == 0.0, 1.0, den)
    y = out / den + bias_ref[...]
    mean = jnp.mean(y, axis=0, keepdims=True)
    var = jnp.mean((y - mean) ** 2, axis=0, keepdims=True)
    yn = (y - mean) / jnp.sqrt(var + 1e-5) * g_ref[...] + b_ref[...]
    o_ref[...] = jnp.where(yn > 0.0, yn, jnp.exp(yn) - 1.0)


_merge = pl.pallas_call(
    _merge_body,
    out_shape=jax.ShapeDtypeStruct((_N, _F), jnp.float32),
)


def kernel(x, edge_index, W_lin, b_lin, W_gat, att_src, att_dst, bias_gat,
           bn_gamma, bn_beta):
    oh = jnp.eye(_H, _D, dtype=jnp.float32)
    ps = (att_src[:, :, None] * oh[:, None, :]).reshape(_F, _D)
    pd = (att_dst[:, :, None] * oh[:, None, :]).reshape(_F, _D)
    sexp = (jnp.arange(_F)[None, :] // _D ==
            jnp.arange(_D)[:, None]).astype(jnp.float32)

    hcat, adst = _proj(x, W_gat, ps, pd)
    pout = _sc_agg(edge_index, hcat, adst)
    return _merge(pout, sexp, bias_gat.reshape(1, _F),
                  bn_gamma.reshape(1, _F), bn_beta.reshape(1, _F))


# speedup vs baseline: 1.1427x; 1.0044x over previous
import functools

import jax
import jax.numpy as jnp
from jax import lax
from jax.experimental import pallas as pl
from jax.experimental.pallas import tpu as pltpu
from jax.experimental.pallas import tpu_sc as plsc

_N = 10000
_E = 320000
_F = 128
_H = 8
_D = 16
_FC = _F + 16
_NC = 2
_NS = 16
_NW = _NC * _NS
_EPW = _E // _NW
_B = 80
_NB = _EPW // _B
_NP = 10240
_RPT = _NP // _NS


def _proj_body(x_ref, wg_ref, ps_ref, pd_ref, hcat_ref, ad_ref):
    h = jnp.dot(x_ref[...], wg_ref[...], preferred_element_type=jnp.float32)
    hcat_ref[:, 0:_F] = h
    hcat_ref[:, _F:_FC] = jnp.dot(h, ps_ref[...],
                                  preferred_element_type=jnp.float32)
    ad_ref[...] = jnp.dot(h, pd_ref[...], preferred_element_type=jnp.float32)


_proj = pl.pallas_call(
    _proj_body,
    out_shape=(
        jax.ShapeDtypeStruct((_N, _FC), jnp.float32),
        jax.ShapeDtypeStruct((_N, _D), jnp.float32),
    ),
)


def _sc_body(ei_hbm, hcat_hbm, adst_hbm, pout_hbm,
             idx0, idx1, hrow0, brow0, hrow1, brow1,
             msgw0, acc_sh, semi0, semi1, semg0, semg1):
    cid = lax.axis_index("c")
    sid = lax.axis_index("s")
    wid = sid * _NC + cid
    bufs = ((idx0, hrow0, brow0, msgw0, semi0, semg0),
            (idx1, hrow1, brow1, msgw0, semi1, semg1))

    lane = lax.iota(jnp.int32, 16)
    headmask = lane < _H
    base0 = wid * _EPW

    def fire_idx(it, p):
        base = base0 + it * _B
        pltpu.async_copy(ei_hbm.at[:, pl.ds(base, _B)], bufs[p][0],
                         bufs[p][4])

    def wait_idx(it, p):
        base = base0 + it * _B
        pltpu.make_async_copy(ei_hbm.at[:, pl.ds(base, _B)], bufs[p][0],
                              bufs[p][4]).wait()

    def fire_gather(p):
        pltpu.async_copy(hcat_hbm.at[bufs[p][0].at[0]], bufs[p][1],
                         bufs[p][5])
        pltpu.async_copy(adst_hbm.at[bufs[p][0].at[1]], bufs[p][2],
                         bufs[p][5])

    def wait_gather(p):
        pltpu.make_async_copy(hcat_hbm.at[bufs[p][0].at[0]], bufs[p][1],
                              bufs[p][5]).wait()
        pltpu.make_async_copy(adst_hbm.at[bufs[p][0].at[1]], bufs[p][2],
                              bufs[p][5]).wait()

    splat_idx = [jnp.full((16,), hd, jnp.int32) for hd in range(_H)]

    def compute_scatter(p):
        hrow, brow, msgw = bufs[p][1], bufs[p][2], bufs[p][3]

        @plsc.parallel_loop(0, _B, 1, unroll=4)
        def edge(i):
            a = hrow[i, pl.ds(_F, 16)]
            b = brow[i, :]
            e = a + b
            e = jnp.where(e >= 0.0, e, e * 0.2)
            w = jnp.where(headmask, jnp.exp(e), 0.0)
            msgw[i, pl.ds(_F, 16)] = w
            for hd in range(_H):
                ws = jnp.take_along_axis(w, splat_idx[hd], axis=0)
                msgw[i, pl.ds(hd * 16, 16)] = hrow[i, pl.ds(hd * 16, 16)] * ws

        pltpu.sync_copy(msgw, acc_sh.at[bufs[p][0].at[1]], add=True)

    fire_idx(0, 0)
    wait_idx(0, 0)
    fire_gather(0)
    fire_idx(1, 1)

    zeros16 = jnp.zeros((16,), jnp.float32)

    def zrow(r, carry):
        for j in range(_FC // 16):
            msgw0[r, pl.ds(j * 16, 16)] = zeros16
        return carry

    lax.fori_loop(0, _B, zrow, 0)
    r0 = sid * _RPT
    for k in range(_RPT // _B):
        pltpu.sync_copy(msgw0, acc_sh.at[pl.ds(r0 + k * _B, _B)])
    plsc.subcore_barrier()

    def paired(g, carry):
        ite = 2 * g
        wait_idx(ite + 1, 1)
        fire_gather(1)
        wait_gather(0)
        compute_scatter(0)
        fire_idx(ite + 2, 0)
        wait_idx(ite + 2, 0)
        fire_gather(0)
        wait_gather(1)
        compute_scatter(1)

        @pl.when(g < (_NB - 1) // 2 - 1)
        def _():
            fire_idx(ite + 3, 1)

        return carry

    lax.fori_loop(0, (_NB - 1) // 2, paired, 0)
    wait_gather(0)
    compute_scatter(0)

    plsc.subcore_barrier()
    pltpu.sync_copy(acc_sh.at[pl.ds(r0, _RPT)],
                    pout_hbm.at[cid, pl.ds(r0, _RPT)])


_sc_agg = functools.partial(
    pl.kernel,
    out_type=jax.ShapeDtypeStruct((_NC, _NP, _FC), jnp.float32),
    mesh=plsc.VectorSubcoreMesh(core_axis_name="c", subcore_axis_name="s"),
    compiler_params=pltpu.CompilerParams(use_tc_tiling_on_sc=False),
    scratch_types=[
        pltpu.VMEM((2, _B), jnp.int32),
        pltpu.VMEM((2, _B), jnp.int32),
        pltpu.VMEM((_B, _FC), jnp.float32),
        pltpu.VMEM((_B, _D), jnp.float32),
        pltpu.VMEM((_B, _FC), jnp.float32),
        pltpu.VMEM((_B, _D), jnp.float32),
        pltpu.VMEM((_B, _FC), jnp.float32),
        pltpu.VMEM_SHARED((_NP, _FC), jnp.float32),
        pltpu.SemaphoreType.DMA,
        pltpu.SemaphoreType.DMA,
        pltpu.SemaphoreType.DMA,
        pltpu.SemaphoreType.DMA,
    ],
)(_sc_body)


def _merge_body(p_ref, s_ref, bias_ref, g_ref, b_ref, o_ref):
    pc = p_ref[0, 0:_N, :] + p_ref[1, 0:_N, :]
    out = pc[:, 0:_F]
    den16 = pc[:, _F:_FC]
    den = jnp.dot(den16, s_ref[...], preferred_element_type=jnp.float32)
    den = jnp.where(den == 0.0, 1.0, den)
    y = out / den + bias_ref[...]
    mean = jnp.mean(y, axis=0, keepdims=True)
    var = jnp.mean((y - mean) ** 2, axis=0, keepdims=True)
    yn = (y - mean) / jnp.sqrt(var + 1e-5) * g_ref[...] + b_ref[...]
    o_ref[...] = jnp.where(yn > 0.0, yn, jnp.exp(yn) - 1.0)


_merge = pl.pallas_call(
    _merge_body,
    out_shape=jax.ShapeDtypeStruct((_N, _F), jnp.float32),
)


def kernel(x, edge_index, W_lin, b_lin, W_gat, att_src, att_dst, bias_gat,
           bn_gamma, bn_beta):
    oh = jnp.eye(_H, _D, dtype=jnp.float32)
    ps = (att_src[:, :, None] * oh[:, None, :]).reshape(_F, _D)
    pd = (att_dst[:, :, None] * oh[:, None, :]).reshape(_F, _D)
    sexp = (jnp.arange(_F)[None, :] // _D ==
            jnp.arange(_D)[:, None]).astype(jnp.float32)

    hcat, adst = _proj(x, W_gat, ps, pd)
    pout = _sc_agg(edge_index, hcat, adst)
    return _merge(pout, sexp, bias_gat.reshape(1, _F),
                  bn_gamma.reshape(1, _F), bn_beta.reshape(1, _F))


# speedup vs baseline: 1.3258x; 1.1602x over previous
import functools

import jax
import jax.numpy as jnp
from jax import lax
from jax.experimental import pallas as pl
from jax.experimental.pallas import tpu as pltpu
from jax.experimental.pallas import tpu_sc as plsc

_N = 10000
_E = 320000
_F = 128
_H = 8
_D = 16
_FC = _F + 16
_FX = _F + 32
_NC = 2
_NS = 16
_NW = _NC * _NS
_EPW = _E // _NW
_B = 80
_NB = _EPW // _B
_NP = 10240
_RPT = _NP // _NS


def _proj_body(x_ref, wg_ref, ps_ref, pd_ref, p1_ref, p2_ref, hx_ref, ad_ref):
    h = jnp.dot(x_ref[...], wg_ref[...], preferred_element_type=jnp.float32)
    hx_ref[:, 0:_F] = jnp.dot(
        h, p1_ref[...], preferred_element_type=jnp.float32
    ).astype(jnp.bfloat16)
    a16 = jnp.dot(h, ps_ref[...], preferred_element_type=jnp.float32)
    hx_ref[:, _F:_FX] = jnp.dot(
        a16, p2_ref[...], preferred_element_type=jnp.float32
    ).astype(jnp.bfloat16)
    ad_ref[...] = jnp.dot(h, pd_ref[...], preferred_element_type=jnp.float32)


_proj = pl.pallas_call(
    _proj_body,
    out_shape=(
        jax.ShapeDtypeStruct((_N, _FX), jnp.bfloat16),
        jax.ShapeDtypeStruct((_N, _D), jnp.float32),
    ),
)


def _sc_body(ei_hbm, hx_hbm, adst_hbm, pout_hbm,
             idx0, idx1, hrow0, brow0, hrow1, brow1, msgw0, msgw1,
             sdidx0, sdidx1, acc_sh,
             semi0, semi1, semg0, semg1, sems0, sems1):
    cid = lax.axis_index("c")
    sid = lax.axis_index("s")
    wid = sid * _NC + cid
    bufs = ((idx0, hrow0, brow0, msgw0, semi0, semg0, sdidx0, sems0),
            (idx1, hrow1, brow1, msgw1, semi1, semg1, sdidx1, sems1))

    lane = lax.iota(jnp.int32, 16)
    headmask = lane < _H
    base0 = wid * _EPW

    def fire_idx(it, p):
        base = base0 + it * _B
        pltpu.async_copy(ei_hbm.at[:, pl.ds(base, _B)], bufs[p][0],
                         bufs[p][4])

    def wait_idx(it, p):
        base = base0 + it * _B
        pltpu.make_async_copy(ei_hbm.at[:, pl.ds(base, _B)], bufs[p][0],
                              bufs[p][4]).wait()

    def fire_gather(p):
        pltpu.async_copy(hx_hbm.at[bufs[p][0].at[0]], bufs[p][1],
                         bufs[p][5])
        pltpu.async_copy(adst_hbm.at[bufs[p][0].at[1]], bufs[p][2],
                         bufs[p][5])

    def wait_gather(p):
        pltpu.make_async_copy(hx_hbm.at[bufs[p][0].at[0]], bufs[p][1],
                              bufs[p][5]).wait()
        pltpu.make_async_copy(adst_hbm.at[bufs[p][0].at[1]], bufs[p][2],
                              bufs[p][5]).wait()

    splat_idx = [jnp.full((16,), hd, jnp.int32) for hd in range(_H)]

    def compute(p):
        hrow, brow, msgw = bufs[p][1], bufs[p][2], bufs[p][3]

        @plsc.parallel_loop(0, _B, 1, unroll=4)
        def edge(i):
            av = hrow[i, pl.ds(_F, 32)]
            a, _ = plsc.unpack(av, format=plsc.PackFormat.INTERLEAVED)
            b = brow[i, :]
            e = a + b
            e = jnp.where(e >= 0.0, e, e * 0.2)
            w = jnp.where(headmask, jnp.exp(e), 0.0)
            msgw[i, pl.ds(_F, 16)] = w
            for j in range(_H // 2):
                hv = hrow[i, pl.ds(j * 32, 32)]
                h0, h1 = plsc.unpack(hv, format=plsc.PackFormat.INTERLEAVED)
                w0 = jnp.take_along_axis(w, splat_idx[2 * j], axis=0)
                w1 = jnp.take_along_axis(w, splat_idx[2 * j + 1], axis=0)
                msgw[i, pl.ds(j * 32, 16)] = h0 * w0
                msgw[i, pl.ds(j * 32 + 16, 16)] = h1 * w1

    def fire_scatter(p):
        idx, sdidx = bufs[p][0], bufs[p][6]
        for k in range(_B // 16):
            sdidx[pl.ds(k * 16, 16)] = idx[1, pl.ds(k * 16, 16)]
        pltpu.async_copy(bufs[p][3], acc_sh.at[sdidx], bufs[p][7], add=True)

    def wait_scatter(p):
        pltpu.make_async_copy(bufs[p][3], acc_sh.at[bufs[p][6]],
                              bufs[p][7]).wait()

    fire_idx(0, 0)
    wait_idx(0, 0)
    fire_gather(0)
    fire_idx(1, 1)

    zeros16 = jnp.zeros((16,), jnp.float32)

    def zrow(r, carry):
        for j in range(_FC // 16):
            msgw0[r, pl.ds(j * 16, 16)] = zeros16
        return carry

    lax.fori_loop(0, _B, zrow, 0)
    r0 = sid * _RPT
    for k in range(_RPT // _B):
        pltpu.sync_copy(msgw0, acc_sh.at[pl.ds(r0 + k * _B, _B)])
    plsc.subcore_barrier()

    def paired(g, carry):
        ite = 2 * g
        wait_idx(ite + 1, 1)
        fire_gather(1)
        wait_gather(0)
        compute(0)

        @pl.when(g > 0)
        def _():
            wait_scatter(1)

        fire_scatter(0)
        fire_idx(ite + 2, 0)
        wait_idx(ite + 2, 0)
        fire_gather(0)
        wait_gather(1)
        compute(1)
        wait_scatter(0)
        fire_scatter(1)

        @pl.when(g < (_NB - 1) // 2 - 1)
        def _():
            fire_idx(ite + 3, 1)

        return carry

    lax.fori_loop(0, (_NB - 1) // 2, paired, 0)
    wait_gather(0)
    compute(0)
    wait_scatter(1)
    fire_scatter(0)
    wait_scatter(0)

    plsc.subcore_barrier()
    pltpu.sync_copy(acc_sh.at[pl.ds(r0, _RPT)],
                    pout_hbm.at[cid, pl.ds(r0, _RPT)])


_sc_agg = functools.partial(
    pl.kernel,
    out_type=jax.ShapeDtypeStruct((_NC, _NP, _FC), jnp.float32),
    mesh=plsc.VectorSubcoreMesh(core_axis_name="c", subcore_axis_name="s"),
    compiler_params=pltpu.CompilerParams(use_tc_tiling_on_sc=False,
                                         needs_layout_passes=False),
    scratch_types=[
        pltpu.VMEM((2, _B), jnp.int32),
        pltpu.VMEM((2, _B), jnp.int32),
        pltpu.VMEM((_B, _FX), jnp.bfloat16),
        pltpu.VMEM((_B, _D), jnp.float32),
        pltpu.VMEM((_B, _FX), jnp.bfloat16),
        pltpu.VMEM((_B, _D), jnp.float32),
        pltpu.VMEM((_B, _FC), jnp.float32),
        pltpu.VMEM((_B, _FC), jnp.float32),
        pltpu.VMEM((_B,), jnp.int32),
        pltpu.VMEM((_B,), jnp.int32),
        pltpu.VMEM_SHARED((_NP, _FC), jnp.float32),
        pltpu.SemaphoreType.DMA,
        pltpu.SemaphoreType.DMA,
        pltpu.SemaphoreType.DMA,
        pltpu.SemaphoreType.DMA,
        pltpu.SemaphoreType.DMA,
        pltpu.SemaphoreType.DMA,
    ],
)(_sc_body)


def _merge_body(p_ref, s_ref, bias_ref, g_ref, b_ref, o_ref):
    pc = p_ref[0, 0:_N, :] + p_ref[1, 0:_N, :]
    out = pc[:, 0:_F]
    den16 = pc[:, _F:_FC]
    den = jnp.dot(den16, s_ref[...], preferred_element_type=jnp.float32)
    den = jnp.where(den == 0.0, 1.0, den)
    y = out / den + bias_ref[...]
    mean = jnp.mean(y, axis=0, keepdims=True)
    var = jnp.mean((y - mean) ** 2, axis=0, keepdims=True)
    yn = (y - mean) / jnp.sqrt(var + 1e-5) * g_ref[...] + b_ref[...]
    o_ref[...] = jnp.where(yn > 0.0, yn, jnp.exp(yn) - 1.0)


_merge = pl.pallas_call(
    _merge_body,
    out_shape=jax.ShapeDtypeStruct((_N, _F), jnp.float32),
)


def kernel(x, edge_index, W_lin, b_lin, W_gat, att_src, att_dst, bias_gat,
           bn_gamma, bn_beta):
    oh = jnp.eye(_H, _D, dtype=jnp.float32)
    ps = (att_src[:, :, None] * oh[:, None, :]).reshape(_F, _D)
    pd = (att_dst[:, :, None] * oh[:, None, :]).reshape(_F, _D)
    csrc = jnp.arange(_F)
    cdst = (csrc // 32) * 32 + (csrc % _D) * 2 + (csrc // _D) % 2
    p1 = (jnp.arange(_F)[None, :] == cdst[:, None]).astype(jnp.float32)
    p2 = (jnp.arange(32)[None, :] ==
          (2 * jnp.arange(_D))[:, None]).astype(jnp.float32)
    sexp = (jnp.arange(_F)[None, :] // _D ==
            jnp.arange(_D)[:, None]).astype(jnp.float32)

    hx, adst = _proj(x, W_gat, ps, pd, p1, p2)
    pout = _sc_agg(edge_index, hx, adst)
    return _merge(pout, sexp, bias_gat.reshape(1, _F),
                  bn_gamma.reshape(1, _F), bn_beta.reshape(1, _F))
